# baseline (speedup 1.0000x reference)
import functools

import jax
import jax.numpy as jnp
from jax import lax
from jax.experimental import pallas as pl
from jax.experimental.pallas import tpu as pltpu
from jax.experimental.pallas import tpu_sc as plsc


def _sc_gather(table, idx):
    info = plsc.get_sparse_core_info()
    num_workers = info.num_cores * info.num_subcores
    b = idx.shape[0]
    d = table.shape[1]
    b_per_w = b // num_workers
    mesh = plsc.VectorSubcoreMesh(core_axis_name="c", subcore_axis_name="s")

    @functools.partial(
        pl.kernel,
        mesh=mesh,
        compiler_params=pltpu.CompilerParams(use_tc_tiling_on_sc=False),
        out_type=jax.ShapeDtypeStruct((b, d), jnp.float32),
        scratch_types=[
            pltpu.VMEM((b_per_w,), jnp.int32),
            pltpu.VMEM((b_per_w, d), jnp.float32),
            pltpu.SemaphoreType.DMA,
        ],
    )
    def k(table_hbm, idx_hbm, out_hbm, idx_v, rows_v, sem):
        wid = lax.axis_index("s") * info.num_cores + lax.axis_index("c")
        base = wid * b_per_w
        pltpu.sync_copy(idx_hbm.at[pl.ds(base, b_per_w)], idx_v)
        pltpu.async_copy(table_hbm.at[idx_v], rows_v, sem).wait()
        pltpu.sync_copy(rows_v, out_hbm.at[pl.ds(base, b_per_w)])

    return k(table, idx)


def _tc_xpart(x, y=None, l_tile=8, blk0=0, nblk=None):
    b, l, f = x.shape
    if nblk is None:
        nblk = pl.cdiv(l, l_tile)

    def body(*refs):
        x_ref, out_ref = refs[-2], refs[-1]
        for j in range(l_tile):
            out_ref[j, :, :] = x_ref[:, j, :].T

    in_specs = [pl.BlockSpec((b, l_tile, f), lambda i: (0, i + blk0, 0))]
    args = (x,)
    aliases = {}
    if y is not None:
        in_specs.insert(0, pl.BlockSpec(memory_space=pl.ANY))
        args = (y, x)
        aliases = {0: 0}

    def body2(*refs):
        x_ref, out_ref = refs[-2], refs[-1]
        for j in range(l_tile):
            out_ref[j, :, :] = x_ref[:, j, :].T

    return pl.pallas_call(
        body2,
        grid=(nblk,),
        in_specs=in_specs,
        out_specs=pl.BlockSpec((l_tile, f, b), lambda i: (i + blk0, 0, 0)),
        out_shape=jax.ShapeDtypeStruct((l, f + 64, b), jnp.float32),
        input_output_aliases=aliases,
    )(*args)


def _tc_embpart(y, id_t, po3, l_tile=40):
    l, w, b = y.shape
    e = w - 128

    def body(y_ref, id_ref, po_ref, out_ref):
        del y_ref
        for j in range(l_tile):
            out_ref[j, :, :] = po_ref[j, :, :] + id_ref[...]

    return pl.pallas_call(
        body,
        grid=(pl.cdiv(l, l_tile),),
        in_specs=[
            pl.BlockSpec(memory_space=pl.ANY),
            pl.BlockSpec((e, b), lambda i: (0, 0)),
            pl.BlockSpec((l_tile, e, 1), lambda i: (i, 0, 0)),
        ],
        out_specs=pl.BlockSpec((l_tile, e, b), lambda i: (i, 2, 0)),
        out_shape=jax.ShapeDtypeStruct((l, w, b), jnp.float32),
        input_output_aliases={0: 0},
    )(y, id_t, po3)


def kernel(series_id, x, id_embed, po_embed):
    y1 = _tc_xpart(x, l_tile=8, blk0=0, nblk=13)
    id_rows = _sc_gather(id_embed, series_id.astype(jnp.int32))
    y2 = _tc_xpart(x, y=y1, l_tile=8, blk0=13, nblk=12)
    out_t = _tc_embpart(y2, id_rows.T, po_embed[:, :, None])
    return jnp.transpose(out_t, (2, 0, 1))


# speedup vs baseline: 1.0574x; 1.0574x over previous
import functools

import jax
import jax.numpy as jnp
from jax import lax
from jax.experimental import pallas as pl
from jax.experimental.pallas import tpu as pltpu
from jax.experimental.pallas import tpu_sc as plsc


def _sc_gather(table, idx):
    info = plsc.get_sparse_core_info()
    num_workers = info.num_cores * info.num_subcores
    b = idx.shape[0]
    d = table.shape[1]
    b_per_w = b // num_workers
    mesh = plsc.VectorSubcoreMesh(core_axis_name="c", subcore_axis_name="s")

    @functools.partial(
        pl.kernel,
        mesh=mesh,
        compiler_params=pltpu.CompilerParams(use_tc_tiling_on_sc=False),
        out_type=jax.ShapeDtypeStruct((b, d), jnp.float32),
        scratch_types=[
            pltpu.VMEM((b_per_w,), jnp.int32),
            pltpu.VMEM((b_per_w, d), jnp.float32),
            pltpu.SemaphoreType.DMA,
        ],
    )
    def k(table_hbm, idx_hbm, out_hbm, idx_v, rows_v, sem):
        wid = lax.axis_index("s") * info.num_cores + lax.axis_index("c")
        base = wid * b_per_w
        pltpu.sync_copy(idx_hbm.at[pl.ds(base, b_per_w)], idx_v)
        pltpu.async_copy(table_hbm.at[idx_v], rows_v, sem).wait()
        pltpu.sync_copy(rows_v, out_hbm.at[pl.ds(base, b_per_w)])

    return k(table, idx)


def _tc_fuse_t(x, id_t, po3, l_tile=8):
    b, l, f = x.shape
    e = po3.shape[1]

    def body(x_ref, id_ref, po_ref, out_ref):
        for j in range(l_tile):
            out_ref[j, 0:f, :] = x_ref[:, j, :].T
            out_ref[j, f:, :] = po_ref[j, :, :] + id_ref[...]

    return pl.pallas_call(
        body,
        grid=(l // l_tile,),
        in_specs=[
            pl.BlockSpec((b, l_tile, f), lambda i: (0, i, 0)),
            pl.BlockSpec((e, b), lambda i: (0, 0)),
            pl.BlockSpec((l_tile, e, 1), lambda i: (i, 0, 0)),
        ],
        out_specs=pl.BlockSpec((l_tile, f + e, b), lambda i: (i, 0, 0)),
        out_shape=jax.ShapeDtypeStruct((l, f + e, b), jnp.float32),
    )(x, id_t, po3)


def kernel(series_id, x, id_embed, po_embed):
    id_rows = _sc_gather(id_embed, series_id.astype(jnp.int32))
    out_t = _tc_fuse_t(x, id_rows.T, po_embed[:, :, None])
    return jnp.transpose(out_t, (2, 0, 1))


# speedup vs baseline: 1.0811x; 1.0224x over previous
import functools

import jax
import jax.numpy as jnp
from jax import lax
from jax.experimental import pallas as pl
from jax.experimental.pallas import tpu as pltpu
from jax.experimental.pallas import tpu_sc as plsc


def _sc_gather(table, idx):
    info = plsc.get_sparse_core_info()
    num_workers = info.num_cores * info.num_subcores
    b = idx.shape[0]
    d = table.shape[1]
    b_per_w = b // num_workers
    mesh = plsc.VectorSubcoreMesh(core_axis_name="c", subcore_axis_name="s")

    @functools.partial(
        pl.kernel,
        mesh=mesh,
        compiler_params=pltpu.CompilerParams(use_tc_tiling_on_sc=False),
        out_type=jax.ShapeDtypeStruct((b, d), jnp.float32),
        scratch_types=[
            pltpu.VMEM((b_per_w,), jnp.int32),
            pltpu.VMEM((b_per_w, d), jnp.float32),
            pltpu.SemaphoreType.DMA,
        ],
    )
    def k(table_hbm, idx_hbm, out_hbm, idx_v, rows_v, sem):
        wid = lax.axis_index("s") * info.num_cores + lax.axis_index("c")
        base = wid * b_per_w
        pltpu.sync_copy(idx_hbm.at[pl.ds(base, b_per_w)], idx_v)
        pltpu.async_copy(table_hbm.at[idx_v], rows_v, sem).wait()
        pltpu.sync_copy(rows_v, out_hbm.at[pl.ds(base, b_per_w)])

    return k(table, idx)


def _tc_fuse_t(x, id_t, po3, l_tile=16):
    b, l, f = x.shape
    e = po3.shape[1]

    def body(x_ref, id_ref, po_ref, out_ref):
        for j in range(l_tile):
            out_ref[j, 0:f, :] = x_ref[:, j, :].T
            out_ref[j, f:, :] = po_ref[j, :, :] + id_ref[...]

    return pl.pallas_call(
        body,
        grid=(pl.cdiv(l, l_tile),),
        in_specs=[
            pl.BlockSpec((b, l_tile, f), lambda i: (0, i, 0)),
            pl.BlockSpec((e, b), lambda i: (0, 0)),
            pl.BlockSpec((l_tile, e, 1), lambda i: (i, 0, 0)),
        ],
        out_specs=pl.BlockSpec((l_tile, f + e, b), lambda i: (i, 0, 0)),
        out_shape=jax.ShapeDtypeStruct((l, f + e, b), jnp.float32),
    )(x, id_t, po3)


def kernel(series_id, x, id_embed, po_embed):
    id_rows = _sc_gather(id_embed, series_id.astype(jnp.int32))
    out_t = _tc_fuse_t(x, id_rows.T, po_embed[:, :, None])
    return jnp.transpose(out_t, (2, 0, 1))
